# Initial kernel scaffold; baseline (speedup 1.0000x reference)
import jax
import jax.numpy as jnp
from jax.experimental import pallas as pl


def kernel(table, idx, targets):
    raise NotImplementedError("write your pallas kernel here")


# speedup vs baseline: 1.6211x; 1.6211x over previous
import functools

import jax
import jax.numpy as jnp
from jax import lax
from jax.experimental import pallas as pl
from jax.experimental.pallas import tpu as pltpu
from jax.experimental.pallas import tpu_sc as plsc

C = 1000
CP = 1024
N = 1024 * 50

_info = plsc.get_sparse_core_info()
_NC, _NS, _L = _info.num_cores, _info.num_subcores, _info.num_lanes
_NW = _NC * _NS
_PER_W = N // _NW
_CHUNK = 64
_NCHUNK = _PER_W // _CHUNK


def _lse_body(tab_ref, out_ref):
    x = tab_ref[...]
    m = jnp.max(x, axis=1)
    s = jnp.sum(jnp.exp(x - m[:, None]), axis=1)
    out_ref[...] = m + jnp.log(s)


def _row_lse(table):
    return pl.pallas_call(
        _lse_body,
        out_shape=jax.ShapeDtypeStruct((C,), jnp.float32),
    )(table)


_sc_mesh = plsc.VectorSubcoreMesh(core_axis_name="c", subcore_axis_name="s")


@functools.partial(
    pl.kernel,
    mesh=_sc_mesh,
    out_type=[
        jax.ShapeDtypeStruct((N, C), jnp.float32),
        jax.ShapeDtypeStruct((_NW, _L), jnp.float32),
    ],
    scratch_types=[
        pltpu.VMEM((_CHUNK,), jnp.int32),
        pltpu.VMEM((_CHUNK,), jnp.int32),
        pltpu.VMEM((_CHUNK, C), jnp.float32),
        pltpu.VMEM((C,), jnp.float32),
        pltpu.VMEM((_L,), jnp.float32),
        pltpu.SemaphoreType.DMA,
    ],
    compiler_params=pltpu.CompilerParams(
        needs_layout_passes=False, use_tc_tiling_on_sc=False),
)
def _sc_gather(table_hbm, idx_hbm, tgt_hbm, lse_hbm, out_hbm, part_hbm,
               idx_v, tgt_v, rows_v, lse_v, acc_v, sem):
    wid = lax.axis_index("s") * _NC + lax.axis_index("c")
    base = wid * _PER_W
    pltpu.sync_copy(lse_hbm, lse_v)

    def chunk_body(ci, acc):
        row0 = base + ci * _CHUNK
        pltpu.sync_copy(idx_hbm.at[pl.ds(row0, _CHUNK)], idx_v)
        pltpu.sync_copy(tgt_hbm.at[pl.ds(row0, _CHUNK)], tgt_v)
        pltpu.async_copy(table_hbm.at[idx_v], rows_v, sem).wait()
        pltpu.sync_copy(rows_v, out_hbm.at[pl.ds(row0, _CHUNK)])
        for v in range(_CHUNK // _L):
            j = lax.iota(jnp.int32, _L) + (v * _L)
            idx16 = idx_v[pl.ds(v * _L, _L)]
            tgt16 = tgt_v[pl.ds(v * _L, _L)]
            lse_g = plsc.load_gather(lse_v, [idx16])
            tval = plsc.load_gather(rows_v, [j, tgt16])
            acc = acc + (lse_g - tval)
        return acc

    acc = lax.fori_loop(0, _NCHUNK, chunk_body,
                        jnp.zeros((_L,), jnp.float32))
    acc_v[...] = acc
    pltpu.sync_copy(acc_v, part_hbm.at[wid])


def _loss_body(p_ref, out_ref):
    out_ref[...] = (jnp.sum(p_ref[...]) / N)[None]


def _loss_reduce(partials):
    return pl.pallas_call(
        _loss_body,
        out_shape=jax.ShapeDtypeStruct((1,), jnp.float32),
    )(partials)


def kernel(table, idx, targets):
    lse = _row_lse(table)
    idx_f = idx.reshape(-1)
    tgt_f = targets.reshape(-1)
    logits2d, partials = _sc_gather(table, idx_f, tgt_f, lse)
    loss = _loss_reduce(partials)[0]
    return logits2d, loss
